# Initial kernel scaffold; baseline (speedup 1.0000x reference)
import jax
import jax.numpy as jnp
from jax.experimental import pallas as pl


def kernel(token_ids, summary_mask, table):
    raise NotImplementedError("write your pallas kernel here")


# speedup vs baseline: 2.5298x; 2.5298x over previous
import functools

import jax
import jax.numpy as jnp
from jax import lax
from jax.experimental import pallas as pl
from jax.experimental.pallas import tpu as pltpu
from jax.experimental.pallas import tpu_sc as plsc

_B, _L, _D = 4096, 50, 64
_NC, _NS = 2, 16
_NW = _NC * _NS
_TPW = (_B * _L) // _NW
_RPW = _B // _NW
_CH = 128
_NCH = _TPW // _CH
_G = _D // 16


@functools.partial(
    pl.kernel,
    out_type=(
        jax.ShapeDtypeStruct((_B * _L, _D), jnp.float32),
        jax.ShapeDtypeStruct((_B, _D), jnp.float32),
    ),
    mesh=plsc.VectorSubcoreMesh(core_axis_name="c", subcore_axis_name="s"),
    compiler_params=pltpu.CompilerParams(use_tc_tiling_on_sc=False),
    scratch_types=[
        pltpu.VMEM((_CH,), jnp.int32),
        pltpu.VMEM((_CH,), jnp.float32),
        pltpu.VMEM((_CH, _D), jnp.float32),
        pltpu.VMEM((_RPW, _D), jnp.float32),
        pltpu.VMEM((_RPW, _D), jnp.float32),
        pltpu.SemaphoreType.DMA,
    ],
)
def _glove_sc(tid_hbm, mask_hbm, mpad_hbm, table_hbm, toks_hbm, summ_hbm,
              idx_v, m_v, rows_v, sums_v, mpad_v, sem):
    wid = lax.axis_index("s") * _NC + lax.axis_index("c")
    base_tok = wid * _TPW
    base_row = wid * _RPW

    z16 = jnp.zeros((16,), jnp.float32)

    pltpu.sync_copy(mpad_hbm.at[pl.ds(base_row, _RPW)], mpad_v)

    def zero_row(r, carry):
        for g in range(_G):
            sums_v[r, pl.ds(g * 16, 16)] = z16
        return carry

    lax.fori_loop(0, _RPW, zero_row, 0)

    def chunk_body(c, carry):
        t0 = base_tok + c * _CH
        pltpu.sync_copy(tid_hbm.at[pl.ds(t0, _CH)], idx_v)
        pltpu.sync_copy(mask_hbm.at[pl.ds(t0, _CH)], m_v)
        pltpu.async_copy(table_hbm.at[idx_v], rows_v, sem).wait()
        pltpu.sync_copy(rows_v, toks_hbm.at[pl.ds(t0, _CH)])

        def grp_body(q, inner):
            mv16 = m_v[pl.ds(q * 16, 16)]
            tq = c * _CH + q * 16
            for j in range(16):
                t = q * 16 + j
                b = (tq + j) // _L
                m = mv16[j]
                for g in range(_G):
                    sl = pl.ds(g * 16, 16)
                    sums_v[b, sl] = sums_v[b, sl] + rows_v[t, sl] * m
            return inner

        lax.fori_loop(0, _CH // 16, grp_body, 0)
        return carry

    lax.fori_loop(0, _NCH, chunk_body, 0)

    def finalize(r, carry):
        cnt16 = z16
        for g in range(_G):
            cnt16 = cnt16 + mpad_v[r, pl.ds(g * 16, 16)]
        cnt = cnt16[0]
        for j in range(1, 16):
            cnt = cnt + cnt16[j]
        cnt_vec = jnp.maximum(jax.lax.broadcast(cnt, (16,)), 1.0)
        inv16 = jax.lax.broadcast(1.0, (16,)) / cnt_vec
        for g in range(_G):
            sl = pl.ds(g * 16, 16)
            sums_v[r, sl] = sums_v[r, sl] * inv16
        return carry

    lax.fori_loop(0, _RPW, finalize, 0)
    pltpu.sync_copy(sums_v, summ_hbm.at[pl.ds(base_row, _RPW)])


def kernel(token_ids, summary_mask, table):
    tid = token_ids.reshape(_B * _L).astype(jnp.int32)
    m = summary_mask.astype(jnp.float32)
    mpad = jnp.pad(m, ((0, 0), (0, _D - _L)))
    toks_flat, summary = _glove_sc(tid, m.reshape(_B * _L), mpad, table)
    return summary, toks_flat.reshape(_B, _L, _D)


# speedup vs baseline: 4.4409x; 1.7554x over previous
import functools

import jax
import jax.numpy as jnp
from jax import lax
from jax.experimental import pallas as pl
from jax.experimental.pallas import tpu as pltpu
from jax.experimental.pallas import tpu_sc as plsc

_B, _L, _D = 4096, 50, 64
_NC, _NS = 2, 16
_NW = _NC * _NS
_TPW = (_B * _L) // _NW
_RPW = _B // _NW
_CH = 128
_NCH = _TPW // _CH
_G = _D // 16


@functools.partial(
    pl.kernel,
    out_type=(
        jax.ShapeDtypeStruct((_B * _L, _D), jnp.float32),
        jax.ShapeDtypeStruct((_B, _D), jnp.float32),
    ),
    mesh=plsc.VectorSubcoreMesh(core_axis_name="c", subcore_axis_name="s"),
    compiler_params=pltpu.CompilerParams(use_tc_tiling_on_sc=False),
    scratch_types=[
        pltpu.VMEM((_TPW,), jnp.int32),
        pltpu.VMEM((_TPW,), jnp.float32),
        pltpu.VMEM((_CH, _D), jnp.float32),
        pltpu.VMEM((_CH, _D), jnp.float32),
        pltpu.VMEM((_RPW, _D), jnp.float32),
        pltpu.VMEM((_RPW, _D), jnp.float32),
        pltpu.VMEM((8, _D), jnp.float32),
        pltpu.SemaphoreType.DMA,
        pltpu.SemaphoreType.DMA,
        pltpu.SemaphoreType.DMA,
        pltpu.SemaphoreType.DMA,
    ],
)
def _glove_sc(tid_hbm, mask_hbm, table_hbm, toks_hbm, summ_hbm,
              idx_v, m_v, rows_a, rows_b, sums_v, cnt_v, acc_v,
              gsem_a, gsem_b, osem_a, osem_b):
    wid = lax.axis_index("s") * _NC + lax.axis_index("c")
    base_tok = wid * _TPW
    base_row = wid * _RPW

    z16 = jnp.zeros((16,), jnp.float32)

    pltpu.sync_copy(tid_hbm.at[pl.ds(base_tok, _TPW)], idx_v)
    pltpu.sync_copy(mask_hbm.at[pl.ds(base_tok, _TPW)], m_v)

    def gather_desc(c, rows, sem):
        return pltpu.make_async_copy(
            table_hbm.at[idx_v.at[pl.ds(c * _CH, _CH)]], rows, sem)

    def out_desc(c, rows, sem):
        return pltpu.make_async_copy(
            rows, toks_hbm.at[pl.ds(base_tok + c * _CH, _CH)], sem)

    gather_desc(0, rows_a, gsem_a).start()
    gather_desc(1, rows_b, gsem_b).start()

    def accum_chunk(c, rows, st):
        def grp_body(q, st2):
            l, b = st2
            a0 = acc_v[0, pl.ds(0, 16)]
            a1 = acc_v[1, pl.ds(0, 16)]
            a2 = acc_v[2, pl.ds(0, 16)]
            a3 = acc_v[3, pl.ds(0, 16)]
            c16 = acc_v[4, pl.ds(0, 16)]
            mv16 = m_v[pl.ds(c * _CH + q * 16, 16)]
            for j in range(16):
                is_new = l == 0

                @pl.when(jnp.logical_and(is_new, b >= 0))
                def _flush(a0=a0, a1=a1, a2=a2, a3=a3, c16=c16, b=b):
                    sums_v[b, pl.ds(0, 16)] = a0
                    sums_v[b, pl.ds(16, 16)] = a1
                    sums_v[b, pl.ds(32, 16)] = a2
                    sums_v[b, pl.ds(48, 16)] = a3
                    cnt_v[b, pl.ds(0, 16)] = c16

                b = jnp.where(is_new, b + 1, b)
                m16 = lax.broadcast(mv16[j], (16,))
                t = q * 16 + j
                a0 = jnp.where(is_new, z16, a0) + rows[t, pl.ds(0, 16)] * m16
                a1 = jnp.where(is_new, z16, a1) + rows[t, pl.ds(16, 16)] * m16
                a2 = jnp.where(is_new, z16, a2) + rows[t, pl.ds(32, 16)] * m16
                a3 = jnp.where(is_new, z16, a3) + rows[t, pl.ds(48, 16)] * m16
                c16 = jnp.where(is_new, z16, c16) + m16
                l = jnp.where(l == _L - 1, 0, l + 1)
            acc_v[0, pl.ds(0, 16)] = a0
            acc_v[1, pl.ds(0, 16)] = a1
            acc_v[2, pl.ds(0, 16)] = a2
            acc_v[3, pl.ds(0, 16)] = a3
            acc_v[4, pl.ds(0, 16)] = c16
            return (l, b)

        return lax.fori_loop(0, _CH // 16, grp_body, st)

    def pair_body(i, st):
        ca = 2 * i
        cb = 2 * i + 1
        gather_desc(ca, rows_a, gsem_a).wait()
        oa = out_desc(ca, rows_a, osem_a)
        oa.start()
        st = accum_chunk(ca, rows_a, st)

        @pl.when(i < _NCH // 2 - 1)
        def _refill_a():
            oa.wait()
            gather_desc(ca + 2, rows_a, gsem_a).start()

        gather_desc(cb, rows_b, gsem_b).wait()
        ob = out_desc(cb, rows_b, osem_b)
        ob.start()
        st = accum_chunk(cb, rows_b, st)

        @pl.when(i < _NCH // 2 - 1)
        def _refill_b():
            ob.wait()
            gather_desc(cb + 2, rows_b, gsem_b).start()

        return st

    st = lax.fori_loop(0, _NCH // 2, pair_body,
                       (jnp.int32(0), jnp.int32(-1)))

    l, b = st
    sums_v[b, pl.ds(0, 16)] = acc_v[0, pl.ds(0, 16)]
    sums_v[b, pl.ds(16, 16)] = acc_v[1, pl.ds(0, 16)]
    sums_v[b, pl.ds(32, 16)] = acc_v[2, pl.ds(0, 16)]
    sums_v[b, pl.ds(48, 16)] = acc_v[3, pl.ds(0, 16)]
    cnt_v[b, pl.ds(0, 16)] = acc_v[4, pl.ds(0, 16)]

    out_desc(_NCH - 2, rows_a, osem_a).wait()
    out_desc(_NCH - 1, rows_b, osem_b).wait()

    one16 = z16 + 1.0

    def finalize(r, carry):
        inv16 = one16 / jnp.maximum(cnt_v[r, pl.ds(0, 16)], one16)
        for g in range(_G):
            sl = pl.ds(g * 16, 16)
            sums_v[r, sl] = sums_v[r, sl] * inv16
        return carry

    lax.fori_loop(0, _RPW, finalize, 0)
    pltpu.sync_copy(sums_v, summ_hbm.at[pl.ds(base_row, _RPW)])


def kernel(token_ids, summary_mask, table):
    tid = token_ids.reshape(_B * _L).astype(jnp.int32)
    m = summary_mask.reshape(_B * _L).astype(jnp.float32)
    toks_flat, summary = _glove_sc(tid, m, table)
    return summary, toks_flat.reshape(_B, _L, _D)
